# baseline (speedup 1.0000x reference)
import functools

import jax
import jax.numpy as jnp
from jax import lax
from jax.experimental import pallas as pl
from jax.experimental.pallas import tpu as pltpu
from jax.experimental.pallas import tpu_sc as plsc

B = 4096
L = 50
LP = 56
MP = 64
D = 64
NC = 2
NS = 16
NW = NC * NS
BPW = B // NW
NCHUNK = D // 16


def _pool_kernel(d1_h, m1_h, d2_h, m2_h, tab_h, out_h,
                 idx1_v, idx2_v, m1_v, m2_v,
                 g1a, g2a, g1b, g2b, out_v,
                 s1a, s2a, s1b, s2b):
    wid = lax.axis_index("s") * NC + lax.axis_index("c")
    base = wid * BPW

    pltpu.sync_copy(d1_h.at[pl.ds(base, BPW)], idx1_v)
    pltpu.sync_copy(d2_h.at[pl.ds(base, BPW)], idx2_v)
    pltpu.sync_copy(m1_h.at[pl.ds(base, BPW)], m1_v)
    pltpu.sync_copy(m2_h.at[pl.ds(base, BPW)], m2_v)

    def issue(b, g1, g2, s1, s2):
        pltpu.async_copy(tab_h.at[idx1_v.at[b, pl.ds(0, LP)]], g1, s1)
        pltpu.async_copy(tab_h.at[idx2_v.at[b, pl.ds(0, LP)]], g2, s2)

    def wait(b, g1, g2, s1, s2):
        pltpu.make_async_copy(tab_h.at[idx1_v.at[b, pl.ds(0, LP)]], g1, s1).wait()
        pltpu.make_async_copy(tab_h.at[idx2_v.at[b, pl.ds(0, LP)]], g2, s2).wait()

    def compute(b, g1, g2):
        mrow1 = [m1_v[b, pl.ds(c * 16, 16)] for c in range(MP // 16)]
        mrow2 = [m2_v[b, pl.ds(c * 16, 16)] for c in range(MP // 16)]
        z = jnp.zeros((16,), jnp.float32)
        accs1 = [z] * NCHUNK
        accs2 = [z] * NCHUNK
        for l in range(L):
            c, j = divmod(l, 16)
            mv1 = jnp.full((16,), mrow1[c][j], jnp.float32)
            mv2 = jnp.full((16,), mrow2[c][j], jnp.float32)
            for k in range(NCHUNK):
                accs1[k] = accs1[k] + g1[l, pl.ds(k * 16, 16)] * mv1
                accs2[k] = accs2[k] + g2[l, pl.ds(k * 16, 16)] * mv2
        for k in range(NCHUNK):
            out_v[b, pl.ds(k * 16, 16)] = accs1[k]
        for k in range(NCHUNK):
            out_v[b, pl.ds(D + k * 16, 16)] = accs2[k]

    issue(0, g1a, g2a, s1a, s2a)

    def step(t, carry):
        b0 = 2 * t
        issue(b0 + 1, g1b, g2b, s1b, s2b)
        wait(b0, g1a, g2a, s1a, s2a)
        compute(b0, g1a, g2a)

        @pl.when(t < BPW // 2 - 1)
        def _():
            issue(b0 + 2, g1a, g2a, s1a, s2a)

        wait(b0 + 1, g1b, g2b, s1b, s2b)
        compute(b0 + 1, g1b, g2b)
        return carry

    lax.fori_loop(0, BPW // 2, step, 0)

    pltpu.sync_copy(out_v, out_h.at[pl.ds(base, BPW)])


def _pool(d1, m1, d2, m2, table):
    mesh = plsc.VectorSubcoreMesh(core_axis_name="c", subcore_axis_name="s")
    f = functools.partial(
        pl.kernel,
        out_type=jax.ShapeDtypeStruct((B, 2 * D), jnp.float32),
        mesh=mesh,
        scratch_types=[
            pltpu.VMEM((BPW, LP), jnp.int32),
            pltpu.VMEM((BPW, LP), jnp.int32),
            pltpu.VMEM((BPW, MP), jnp.float32),
            pltpu.VMEM((BPW, MP), jnp.float32),
            pltpu.VMEM((LP, D), jnp.float32),
            pltpu.VMEM((LP, D), jnp.float32),
            pltpu.VMEM((LP, D), jnp.float32),
            pltpu.VMEM((LP, D), jnp.float32),
            pltpu.VMEM((BPW, 2 * D), jnp.float32),
            pltpu.SemaphoreType.DMA,
            pltpu.SemaphoreType.DMA,
            pltpu.SemaphoreType.DMA,
            pltpu.SemaphoreType.DMA,
        ],
        compiler_params=pltpu.CompilerParams(use_tc_tiling_on_sc=False),
    )(_pool_kernel)
    return f(d1, m1, d2, m2, table)


def _mlp_body(opt_ref, w1_ref, b1_ref, w2_ref, b2_ref, o_ref):
    opt = opt_ref[...] * (1.0 / L)
    h = jnp.dot(opt, w1_ref[...], preferred_element_type=jnp.float32)
    h = h + b1_ref[...]
    o = jnp.dot(h, w2_ref[...], preferred_element_type=jnp.float32)
    o_ref[...] = o + b2_ref[...]


def kernel(data_x1, mask_x1, data_x2, mask_x2, table, W1, b1, W2, b2):
    d1 = jnp.pad(data_x1.astype(jnp.int32), ((0, 0), (0, LP - L)))
    d2 = jnp.pad(data_x2.astype(jnp.int32), ((0, 0), (0, LP - L)))
    m1 = jnp.pad(mask_x1, ((0, 0), (0, MP - L)))
    m2 = jnp.pad(mask_x2, ((0, 0), (0, MP - L)))
    opt = _pool(d1, m1, d2, m2, table)
    return pl.pallas_call(
        _mlp_body,
        out_shape=jax.ShapeDtypeStruct((B, 2), jnp.float32),
    )(opt, W1, b1.reshape(1, -1), W2, b2.reshape(1, -1))


# speedup vs baseline: 1.0003x; 1.0003x over previous
import functools

import jax
import jax.numpy as jnp
from jax import lax
from jax.experimental import pallas as pl
from jax.experimental.pallas import tpu as pltpu
from jax.experimental.pallas import tpu_sc as plsc

B = 4096
L = 50
LP = 56
D = 64
NC = 2
NS = 16
NW = NC * NS
BPW = B // NW
NCHUNK = D // 16
NBUF = 4


def _pool_kernel(d_h, m_h, tab_h, out_h,
                 idx_v, m_v, g0, g1, g2, g3, out_v,
                 s0, s1, s2, s3):
    wid = lax.axis_index("s") * NC + lax.axis_index("c")
    base = wid * BPW
    gs = (g0, g1, g2, g3)
    ss = (s0, s1, s2, s3)

    pltpu.sync_copy(d_h.at[pl.ds(base, BPW)], idx_v)
    pltpu.sync_copy(m_h.at[pl.ds(base, BPW)], m_v)

    def issue(b, g, s):
        pltpu.async_copy(tab_h.at[idx_v.at[b, pl.ds(0, 2 * LP)]], g, s)

    def wait(b, g, s):
        pltpu.make_async_copy(tab_h.at[idx_v.at[b, pl.ds(0, 2 * LP)]], g, s).wait()

    def compute(b, g):
        mrow = [m_v[b, pl.ds(c * 16, 16)] for c in range(2 * D // 16)]
        z = jnp.zeros((16,), jnp.float32)
        accs1 = [z] * NCHUNK
        accs2 = [z] * NCHUNK
        for l in range(L):
            c, j = divmod(l, 16)
            mv1 = jnp.full((16,), mrow[c][j], jnp.float32)
            mv2 = jnp.full((16,), mrow[NCHUNK + c][j], jnp.float32)
            for k in range(NCHUNK):
                accs1[k] = accs1[k] + g[l, pl.ds(k * 16, 16)] * mv1
                accs2[k] = accs2[k] + g[LP + l, pl.ds(k * 16, 16)] * mv2
        for k in range(NCHUNK):
            out_v[b, pl.ds(k * 16, 16)] = accs1[k]
        for k in range(NCHUNK):
            out_v[b, pl.ds(D + k * 16, 16)] = accs2[k]

    for j in range(NBUF):
        issue(j, gs[j], ss[j])

    def step(t, carry):
        for j in range(NBUF):
            b = t * NBUF + j
            wait(b, gs[j], ss[j])
            compute(b, gs[j])

            @pl.when(b + NBUF < BPW)
            def _():
                issue(b + NBUF, gs[j], ss[j])

        return carry

    lax.fori_loop(0, BPW // NBUF, step, 0)

    pltpu.sync_copy(out_v, out_h.at[pl.ds(base, BPW)])


def _pool(d, m, table):
    mesh = plsc.VectorSubcoreMesh(core_axis_name="c", subcore_axis_name="s")
    f = functools.partial(
        pl.kernel,
        out_type=jax.ShapeDtypeStruct((B, 2 * D), jnp.float32),
        mesh=mesh,
        scratch_types=[
            pltpu.VMEM((BPW, 2 * LP), jnp.int32),
            pltpu.VMEM((BPW, 2 * D), jnp.float32),
            pltpu.VMEM((2 * LP, D), jnp.float32),
            pltpu.VMEM((2 * LP, D), jnp.float32),
            pltpu.VMEM((2 * LP, D), jnp.float32),
            pltpu.VMEM((2 * LP, D), jnp.float32),
            pltpu.VMEM((BPW, 2 * D), jnp.float32),
            pltpu.SemaphoreType.DMA,
            pltpu.SemaphoreType.DMA,
            pltpu.SemaphoreType.DMA,
            pltpu.SemaphoreType.DMA,
        ],
        compiler_params=pltpu.CompilerParams(use_tc_tiling_on_sc=False),
    )(_pool_kernel)
    return f(d, m, table)


def _mlp_body(opt_ref, w1_ref, b1_ref, w2_ref, b2_ref, o_ref):
    opt = opt_ref[...] * (1.0 / L)
    h = jnp.dot(opt, w1_ref[...], preferred_element_type=jnp.float32)
    h = h + b1_ref[...]
    o = jnp.dot(h, w2_ref[...], preferred_element_type=jnp.float32)
    o_ref[...] = o + b2_ref[...]


def kernel(data_x1, mask_x1, data_x2, mask_x2, table, W1, b1, W2, b2):
    d1 = jnp.pad(data_x1.astype(jnp.int32), ((0, 0), (0, LP - L)))
    d2 = jnp.pad(data_x2.astype(jnp.int32), ((0, 0), (0, LP - L)))
    d = jnp.concatenate((d1, d2), axis=1)
    m1 = jnp.pad(mask_x1, ((0, 0), (0, D - L)))
    m2 = jnp.pad(mask_x2, ((0, 0), (0, D - L)))
    m = jnp.concatenate((m1, m2), axis=1)
    opt = _pool(d, m, table)
    return pl.pallas_call(
        _mlp_body,
        out_shape=jax.ShapeDtypeStruct((B, 2), jnp.float32),
    )(opt, W1, b1.reshape(1, -1), W2, b2.reshape(1, -1))


# speedup vs baseline: 2.2295x; 2.2287x over previous
import functools

import jax
import jax.numpy as jnp
from jax import lax
from jax.experimental import pallas as pl
from jax.experimental.pallas import tpu as pltpu
from jax.experimental.pallas import tpu_sc as plsc

B = 4096
L = 50
LP = 56
D = 64
NC = 2
NS = 16
NW = NC * NS
BPW = B // NW
NCHUNK = D // 16
NBUF = 4


def _pool_kernel(d_h, m_h, tab_h, out_h,
                 idx_v, m_v, g0, g1, g2, g3, out_v,
                 s0, s1, s2, s3):
    wid = lax.axis_index("s") * NC + lax.axis_index("c")
    base = wid * BPW
    gs = (g0, g1, g2, g3)
    ss = (s0, s1, s2, s3)

    pltpu.sync_copy(d_h.at[pl.ds(base, BPW)], idx_v)
    pltpu.sync_copy(m_h.at[pl.ds(base, BPW)], m_v)

    def issue(b, g, s):
        pltpu.async_copy(tab_h.at[idx_v.at[b, pl.ds(0, 2 * LP)]], g, s)

    def wait(b, g, s):
        pltpu.make_async_copy(tab_h.at[idx_v.at[b, pl.ds(0, 2 * LP)]], g, s).wait()

    def compute(b, g):
        mrow = [m_v[b, pl.ds(c * 16, 16)] for c in range(2 * D // 16)]
        z = jnp.zeros((16,), jnp.float32)
        accs1 = [z] * NCHUNK
        accs2 = [z] * NCHUNK
        for l in range(L):
            c, j = divmod(l, 16)
            mv1 = jnp.full((16,), mrow[c][j], jnp.float32)
            mv2 = jnp.full((16,), mrow[NCHUNK + c][j], jnp.float32)
            for k in range(NCHUNK):
                accs1[k] = accs1[k] + g[l, pl.ds(k * 16, 16)] * mv1
                accs2[k] = accs2[k] + g[LP + l, pl.ds(k * 16, 16)] * mv2
        for k in range(NCHUNK):
            out_v[b, pl.ds(k * 16, 16)] = accs1[k]
        for k in range(NCHUNK):
            out_v[b, pl.ds(D + k * 16, 16)] = accs2[k]

    for j in range(NBUF):
        issue(j, gs[j], ss[j])

    def step(t, carry):
        for j in range(NBUF):
            b = t * NBUF + j
            wait(b, gs[j], ss[j])
            compute(b, gs[j])

            @pl.when(b + NBUF < BPW)
            def _():
                issue(b + NBUF, gs[j], ss[j])

        return carry

    lax.fori_loop(0, BPW // NBUF, step, 0)

    pltpu.sync_copy(out_v, out_h.at[pl.ds(base, BPW)])


def _pool(d, m, table):
    mesh = plsc.VectorSubcoreMesh(core_axis_name="c", subcore_axis_name="s")
    f = functools.partial(
        pl.kernel,
        out_type=jax.ShapeDtypeStruct((B, 2 * D), jnp.float32),
        mesh=mesh,
        scratch_types=[
            pltpu.VMEM((BPW, 2 * LP), jnp.int32),
            pltpu.VMEM((BPW, 2 * D), jnp.float32),
            pltpu.VMEM((2 * LP, D), jnp.float32),
            pltpu.VMEM((2 * LP, D), jnp.float32),
            pltpu.VMEM((2 * LP, D), jnp.float32),
            pltpu.VMEM((2 * LP, D), jnp.float32),
            pltpu.VMEM((BPW, 2 * D), jnp.float32),
            pltpu.SemaphoreType.DMA,
            pltpu.SemaphoreType.DMA,
            pltpu.SemaphoreType.DMA,
            pltpu.SemaphoreType.DMA,
        ],
        compiler_params=pltpu.CompilerParams(use_tc_tiling_on_sc=False),
    )(_pool_kernel)
    return f(d, m, table)


def _mlp_body(opt_ref, w1_ref, b1_ref, w2_ref, b2_ref, o_ref):
    opt = opt_ref[...] * (1.0 / L)
    h = jnp.dot(opt, w1_ref[...], preferred_element_type=jnp.float32)
    h = h + b1_ref[...]
    o = jnp.dot(h, w2_ref[...], preferred_element_type=jnp.float32)
    o_ref[...] = o + b2_ref[...]


def kernel(data_x1, mask_x1, data_x2, mask_x2, table, W1, b1, W2, b2):
    iota_b = jnp.arange(B, dtype=jnp.int32)[:, None]
    iota_j = jnp.arange(LP - L, dtype=jnp.int32)[None, :]
    pad1 = (iota_b * 2 * (LP - L) + iota_j) % 1000000
    pad2 = (iota_b * 2 * (LP - L) + (LP - L) + iota_j) % 1000000
    d1 = jnp.concatenate((data_x1.astype(jnp.int32), pad1), axis=1)
    d2 = jnp.concatenate((data_x2.astype(jnp.int32), pad2), axis=1)
    d = jnp.concatenate((d1, d2), axis=1)
    m1 = jnp.pad(mask_x1, ((0, 0), (0, D - L)))
    m2 = jnp.pad(mask_x2, ((0, 0), (0, D - L)))
    m = jnp.concatenate((m1, m2), axis=1)
    opt = _pool(d, m, table)
    return pl.pallas_call(
        _mlp_body,
        out_shape=jax.ShapeDtypeStruct((B, 2), jnp.float32),
    )(opt, W1, b1.reshape(1, -1), W2, b2.reshape(1, -1))
